# baseline (speedup 1.0000x reference)
import functools

import jax
import jax.numpy as jnp
from jax import lax
from jax.experimental import pallas as pl
from jax.experimental.pallas import tpu as pltpu
from jax.experimental.pallas import tpu_sc as plsc

_INFO = plsc.get_sparse_core_info()
_NC = _INFO.num_cores
_NS = _INFO.num_subcores
_NW = _NC * _NS
_CHUNK = 128
_L = 16


def kernel(time, table):
    B = time.shape[0]
    V, D = table.shape
    assert D == _L and B % (_NW * _CHUNK) == 0
    b_per_w = B // _NW
    n_st = (b_per_w * D) // _CHUNK
    spw = b_per_w // _CHUNK

    table_t = table.T.reshape(-1)
    mesh = plsc.VectorSubcoreMesh(core_axis_name="c", subcore_axis_name="s")

    @functools.partial(
        pl.kernel,
        mesh=mesh,
        out_type=jax.ShapeDtypeStruct((D, B), jnp.float32),
        scratch_types=[
            pltpu.VMEM((b_per_w,), jnp.int32),
            pltpu.VMEM((n_st, _CHUNK), jnp.int32),
            pltpu.VMEM((D, b_per_w), jnp.float32),
            pltpu.SemaphoreType.DMA,
            pltpu.SemaphoreType.DMA,
            pltpu.SemaphoreType.DMA,
        ],
        compiler_params=pltpu.CompilerParams(use_tc_tiling_on_sc=False),
    )
    def gather_k(time_hbm, table_hbm, out_hbm, idx_v, il_v, o_v, sg0, sg1, so):
        wid = lax.axis_index("s") * _NC + lax.axis_index("c")
        base = wid * b_per_w
        sems = (sg0, sg1)
        pltpu.sync_copy(time_hbm.at[pl.ds(base, b_per_w)], idx_v)
        copies = {}

        def build_and_fire(k):
            for h in range(spw):
                for t in range(_CHUNK // _L):
                    v = idx_v[pl.ds(h * _CHUNK + t * _L, _L)]
                    il_v[k * spw + h, pl.ds(t * _L, _L)] = v + (k * V)
            copies[k] = [
                pltpu.async_copy(
                    table_hbm.at[il_v.at[k * spw + h]],
                    o_v.at[k, pl.ds(h * _CHUNK, _CHUNK)],
                    sems[k % 2],
                )
                for h in range(spw)
            ]

        build_and_fire(0)
        build_and_fire(1)
        outs = []
        for k in range(D):
            for cpy in copies[k]:
                cpy.wait()
            if k + 2 < D:
                build_and_fire(k + 2)
            outs.append(
                pltpu.async_copy(
                    o_v.at[k], out_hbm.at[k, pl.ds(base, b_per_w)], so
                )
            )
        for cpy in outs:
            cpy.wait()

    return gather_k(time, table_t).T


# speedup vs baseline: 1.0577x; 1.0577x over previous
import functools

import jax
import jax.numpy as jnp
from jax import lax
from jax.experimental import pallas as pl
from jax.experimental.pallas import tpu as pltpu
from jax.experimental.pallas import tpu_sc as plsc

_INFO = plsc.get_sparse_core_info()
_NC = _INFO.num_cores
_NS = _INFO.num_subcores
_NW = _NC * _NS
_CHUNK = 128
_L = 16


def kernel(time, table):
    B = time.shape[0]
    V, D = table.shape
    assert D == _L and B % (_NW * _CHUNK) == 0
    b_per_w = B // _NW
    n_st = (b_per_w * D) // _CHUNK
    spw = b_per_w // _CHUNK

    table_t = table.T.reshape(-1)
    mesh = plsc.VectorSubcoreMesh(core_axis_name="c", subcore_axis_name="s")

    @functools.partial(
        pl.kernel,
        mesh=mesh,
        out_type=jax.ShapeDtypeStruct((D, B), jnp.float32),
        scratch_types=[
            pltpu.VMEM((b_per_w,), jnp.int32),
            pltpu.VMEM((n_st, _CHUNK), jnp.int32),
            pltpu.VMEM((D, b_per_w), jnp.float32),
            pltpu.SemaphoreType.DMA,
            pltpu.SemaphoreType.DMA,
        ],
        compiler_params=pltpu.CompilerParams(use_tc_tiling_on_sc=False),
    )
    def gather_k(time_hbm, table_hbm, out_hbm, idx_v, il_v, o_v, sg0, so):
        wid = lax.axis_index("s") * _NC + lax.axis_index("c")
        base = wid * b_per_w
        pltpu.sync_copy(time_hbm.at[pl.ds(base, b_per_w)], idx_v)
        copies = []
        for h in range(spw):
            for t in range(_CHUNK // _L):
                v = idx_v[pl.ds(h * _CHUNK + t * _L, _L)]
                for k in range(D):
                    il_v[k * spw + h, pl.ds(t * _L, _L)] = v + (k * V)
            copies += [
                pltpu.async_copy(
                    table_hbm.at[il_v.at[k * spw + h]],
                    o_v.at[k, pl.ds(h * _CHUNK, _CHUNK)],
                    sg0,
                )
                for k in range(D)
            ]
        for cpy in copies:
            cpy.wait()
        outs = [
            pltpu.async_copy(o_v.at[k], out_hbm.at[k, pl.ds(base, b_per_w)], so)
            for k in range(D)
        ]
        for cpy in outs:
            cpy.wait()

    return gather_k(time, table_t).T
